# baseline (speedup 1.0000x reference)
import functools

import jax
import jax.numpy as jnp
import numpy as np
from jax.experimental import pallas as pl
from jax.experimental.pallas import tpu as pltpu

B, D, C, P = 2048, 128, 200, 200
N = B + P
TILE = 256
NP_ = 2304
NT = NP_ // TILE
MARGIN = 1.0
N_PAIRS = float(N * (N - 1) // 2)

_PAIRS = np.array([(i, j) for i in range(NT) for j in range(i, NT)],
                  dtype=np.int32).T
NUM_TILES = _PAIRS.shape[1]


def _argmax_body(lab_ref, out_ref):
    v = lab_ref[...]
    m = jnp.max(v, axis=1, keepdims=True)
    iota = jax.lax.broadcasted_iota(jnp.int32, v.shape, 1)
    idx = jnp.min(jnp.where(v == m, iota, C), axis=1, keepdims=True)
    out_ref[...] = idx


def _loss_body(tiles_ref, xi_ref, xjt_ref, li_ref, lj_ref, out_ref):
    t = pl.program_id(0)

    @pl.when(t == 0)
    def _init():
        out_ref[0, 0] = 0.0

    bi = tiles_ref[0, t]
    bj = tiles_ref[1, t]

    xi = xi_ref[...]
    xjt = xjt_ref[...]
    dot = jax.lax.dot_general(xi, xjt, (((1,), (0,)), ((), ())),
                              preferred_element_type=jnp.float32)
    sqi = jnp.sum(xi * xi, axis=1, keepdims=True)
    sqj = jnp.sum(xjt * xjt, axis=0, keepdims=True)
    d2 = jnp.maximum(sqi + sqj - 2.0 * dot, 0.0)

    same = li_ref[...] == lj_ref[...]
    gi = bi * TILE + jax.lax.broadcasted_iota(jnp.int32, (TILE, 1), 0)
    gj = bj * TILE + jax.lax.broadcasted_iota(jnp.int32, (1, TILE), 1)
    valid = (gi < gj) & (gj < N)

    r = jnp.maximum(MARGIN - jnp.sqrt(d2), 0.0)
    neg_val = r * r
    contrib = jnp.where(valid, jnp.where(same, d2, neg_val), 0.0)
    out_ref[0, 0] += jnp.sum(contrib)

    @pl.when(t == NUM_TILES - 1)
    def _finish():
        out_ref[0, 0] = out_ref[0, 0] * (1.0 / N_PAIRS)


@functools.partial(jax.jit)
def kernel(embeddings, labels, prototypes, proto_keys):
    labels_p = jnp.pad(labels, ((0, 0), (0, 256 - C)),
                       constant_values=-np.inf)
    emb_p = jnp.concatenate(
        [embeddings, prototypes,
         jnp.zeros((NP_ - N, D), dtype=embeddings.dtype)], axis=0)
    emb_t = emb_p.T

    lab_col = pl.pallas_call(
        _argmax_body,
        grid=(B // TILE,),
        in_specs=[pl.BlockSpec((TILE, 256), lambda i: (i, 0))],
        out_specs=pl.BlockSpec((TILE, 1), lambda i: (i, 0)),
        out_shape=jax.ShapeDtypeStruct((B, 1), jnp.int32),
    )(labels_p)

    lab_all = jnp.concatenate(
        [lab_col[:, 0], proto_keys.astype(jnp.int32),
         jnp.full((NP_ - N,), -1, jnp.int32)])
    lab_c = lab_all[:, None]
    lab_r = lab_all[None, :]

    tiles = jnp.asarray(_PAIRS)

    out = pl.pallas_call(
        _loss_body,
        grid_spec=pltpu.PrefetchScalarGridSpec(
            num_scalar_prefetch=1,
            grid=(NUM_TILES,),
            in_specs=[
                pl.BlockSpec((TILE, D), lambda t, tiles: (tiles[0, t], 0)),
                pl.BlockSpec((D, TILE), lambda t, tiles: (0, tiles[1, t])),
                pl.BlockSpec((TILE, 1), lambda t, tiles: (tiles[0, t], 0)),
                pl.BlockSpec((1, TILE), lambda t, tiles: (0, tiles[1, t])),
            ],
            out_specs=pl.BlockSpec(memory_space=pltpu.SMEM),
        ),
        out_shape=jax.ShapeDtypeStruct((1, 1), jnp.float32),
    )(tiles, emb_p, emb_t, lab_c, lab_r)
    return out[0, 0]


# speedup vs baseline: 1.5968x; 1.5968x over previous
import jax
import jax.numpy as jnp
import numpy as np
from jax.experimental import pallas as pl
from jax.experimental.pallas import tpu as pltpu

B, D, C, P = 2048, 128, 200, 200
N = B + P
TILE = 768
NP_ = 2304
NT = NP_ // TILE
MARGIN = 1.0
N_PAIRS = float(N * (N - 1) // 2)

_PAIRS = np.array([(i, j) for i in range(NT) for j in range(i, NT)],
                  dtype=np.int32).T
NUM_TILES = _PAIRS.shape[1]

_PAD_VALS = (100.0 * (np.arange(NP_ - N, dtype=np.float32) + 1.0))
_PAD_EMB = np.broadcast_to(_PAD_VALS[:, None], (NP_ - N, D)).copy()
_PAD_LAB = (-(np.arange(NP_ - N, dtype=np.int32) + 1))


def _argmax_body(lab_ref, out_ref):
    v = lab_ref[...]
    m = jnp.max(v, axis=1, keepdims=True)
    iota = jax.lax.broadcasted_iota(jnp.int32, v.shape, 1)
    idx = jnp.min(jnp.where(v == m, iota, C), axis=1, keepdims=True)
    out_ref[...] = idx


def _loss_body(tiles_ref, xi_ref, xjt_ref, li_ref, lj_ref, out_ref):
    t = pl.program_id(0)

    @pl.when(t == 0)
    def _init():
        out_ref[0, 0] = 0.0

    xi = xi_ref[...]
    xjt = xjt_ref[...]
    dot = jax.lax.dot_general(xi, xjt, (((1,), (0,)), ((), ())),
                              preferred_element_type=jnp.float32)
    sqi = jnp.sum(xi * xi, axis=1, keepdims=True)
    sqj = jnp.sum(xjt * xjt, axis=0, keepdims=True)
    d2 = jnp.maximum(sqi + sqj - 2.0 * dot, 0.0)

    same = li_ref[...] == lj_ref[...]
    r = jnp.maximum(MARGIN - jnp.sqrt(d2), 0.0)
    base = jnp.where(same, d2, r * r)

    diag = tiles_ref[0, t] == tiles_ref[1, t]

    @pl.when(diag)
    def _diag():
        gi = jax.lax.broadcasted_iota(jnp.int32, (TILE, 1), 0)
        gj = jax.lax.broadcasted_iota(jnp.int32, (1, TILE), 1)
        out_ref[0, 0] += jnp.sum(jnp.where(gi < gj, base, 0.0))

    @pl.when(jnp.logical_not(diag))
    def _offdiag():
        out_ref[0, 0] += jnp.sum(base)

    @pl.when(t == NUM_TILES - 1)
    def _finish():
        out_ref[0, 0] = out_ref[0, 0] * (1.0 / N_PAIRS)


def kernel(embeddings, labels, prototypes, proto_keys):
    labels_p = jnp.pad(labels, ((0, 0), (0, 256 - C)),
                       constant_values=-np.inf)
    emb_p = jnp.concatenate(
        [embeddings, prototypes, jnp.asarray(_PAD_EMB)], axis=0)
    emb_t = emb_p.T

    lab_col = pl.pallas_call(
        _argmax_body,
        grid=(B // 256,),
        in_specs=[pl.BlockSpec((256, 256), lambda i: (i, 0))],
        out_specs=pl.BlockSpec((256, 1), lambda i: (i, 0)),
        out_shape=jax.ShapeDtypeStruct((B, 1), jnp.int32),
    )(labels_p)

    lab_all = jnp.concatenate(
        [lab_col[:, 0], proto_keys.astype(jnp.int32), jnp.asarray(_PAD_LAB)])
    lab_c = lab_all[:, None]
    lab_r = lab_all[None, :]

    tiles = jnp.asarray(_PAIRS)

    out = pl.pallas_call(
        _loss_body,
        grid_spec=pltpu.PrefetchScalarGridSpec(
            num_scalar_prefetch=1,
            grid=(NUM_TILES,),
            in_specs=[
                pl.BlockSpec((TILE, D), lambda t, tiles: (tiles[0, t], 0)),
                pl.BlockSpec((D, TILE), lambda t, tiles: (0, tiles[1, t])),
                pl.BlockSpec((TILE, 1), lambda t, tiles: (tiles[0, t], 0)),
                pl.BlockSpec((1, TILE), lambda t, tiles: (0, tiles[1, t])),
            ],
            out_specs=pl.BlockSpec(memory_space=pltpu.SMEM),
        ),
        out_shape=jax.ShapeDtypeStruct((1, 1), jnp.float32),
    )(tiles, emb_p, emb_t, lab_c, lab_r)
    return out[0, 0]
